# Initial kernel scaffold; baseline (speedup 1.0000x reference)
import jax
import jax.numpy as jnp
from jax.experimental import pallas as pl


def kernel(hidden_states, attention_mask, layer_emb, cent_emb, spatial_emb, ln_g, ln_b, Wq, bq, Wk, bk, Wv, bv, Wo, bo, position):
    raise NotImplementedError("write your pallas kernel here")


# speedup vs baseline: 1.3800x; 1.3800x over previous
import functools

import jax
import jax.numpy as jnp
import numpy as np
from jax.experimental import pallas as pl

E = 16
D = 768
H = 3
DH = D // H
NCTX = 3
MAXDIST = E
SCALE = 0.01
LANE = 128
HL = H * LANE

TB = 512


def _prep_kernel(le, ce, wqt, bqt, wk, bk, wv, bv, wo, spbt,
                 wspt_ref, bspt_ref, wvop_ref):
    node = le[...] + ce[...]
    kmat = jnp.dot(node, wk[...], preferred_element_type=jnp.float32) + bk[...]
    vmat = jnp.dot(node, wv[...], preferred_element_type=jnp.float32) + bv[...]
    inv = np.float32(1.0 / np.sqrt(DH))
    zrow = jnp.zeros((LANE - E, D), dtype=jnp.float32)
    zcol1 = jnp.full((LANE - E, 1), -1e30, dtype=jnp.float32)
    zvo = jnp.zeros((LANE - E, E), dtype=jnp.float32)
    wspt_parts, bspt_parts, wvop_parts = [], [], []
    for h in range(H):
        sl = slice(h * DH, (h + 1) * DH)
        kh = kmat[:, sl]
        wst_h = jnp.dot(kh, wqt[sl, :], preferred_element_type=jnp.float32) * inv
        bst_h = jnp.dot(kh, bqt[sl, :], preferred_element_type=jnp.float32) * inv
        bst_h = bst_h + spbt[...]
        wvo_h = jnp.dot(vmat[:, sl], wo[sl, :],
                        preferred_element_type=jnp.float32)
        wspt_parts.append(jnp.concatenate([wst_h, zrow], axis=0))
        bspt_parts.append(jnp.concatenate([bst_h, zcol1], axis=0))
        wvop_parts.append(jnp.concatenate([wvo_h, zvo], axis=0))
    wspt_ref[...] = jnp.concatenate(wspt_parts, axis=0)
    bspt_ref[...] = jnp.concatenate(bspt_parts, axis=0)
    wvop_ref[...] = jnp.concatenate(wvop_parts, axis=0)


def _ln(x, g, b):
    mu = jnp.mean(x, axis=-1, keepdims=True)
    xc = x - mu
    var = jnp.mean(xc * xc, axis=-1, keepdims=True)
    return xc * jax.lax.rsqrt(var + 1e-5) * g + b


def _headwise_softmax(s):
    parts = []
    for h in range(H):
        sh = s[:, h * LANE:(h + 1) * LANE]
        mx = jnp.max(sh, axis=-1, keepdims=True)
        eh = jnp.exp(sh - mx)
        parts.append(eh / jnp.sum(eh, axis=-1, keepdims=True))
    return jnp.concatenate(parts, axis=1)


def _main_kernel(hs, mcol, wsp, bsp, ctx, g, b, a_ref):
    j = pl.program_id(1)
    x = hs[0]
    hln = _ln(x, g[...], b[...])
    s = jnp.dot(hln, wsp[...], preferred_element_type=jnp.float32) + bsp[...]
    p = _headwise_softmax(s)
    m = mcol[0, 0]
    partial = jnp.sum(p * m, axis=0, keepdims=True)

    @pl.when(j == 0)
    def _():
        hc = _ln(ctx[...], g[...], b[...])
        sc = jnp.dot(hc, wsp[...], preferred_element_type=jnp.float32) + bsp[...]
        pc = _headwise_softmax(sc)
        a_ref[0] = np.float32(NCTX) * pc + partial

    @pl.when(j != 0)
    def _():
        a_ref[0] = a_ref[0] + partial


def _final_kernel(a, mask, wvop, bo, out_ref):
    msum = jnp.sum(mask[...], axis=-1, keepdims=True) + np.float32(NCTX)
    msum = jnp.maximum(msum, 1e-6)
    acc = jnp.dot(a[...], wvop[...], preferred_element_type=jnp.float32)
    out_ref[...] = acc * (np.float32(SCALE) / msum) + bo[...] * np.float32(SCALE)


@functools.partial(jax.jit, static_argnames=())
def _run(hidden_states, attention_mask, layer_emb, cent_emb, spatial_emb,
         ln_g, ln_b, Wq, bq, Wk, bk, Wv, bv, Wo, bo, position):
    B, S, _ = hidden_states.shape
    nblk = S // TB

    ctx_row = layer_emb[position][None, :]
    dist = jnp.clip(jnp.abs(position - jnp.arange(E)), 0, MAXDIST)
    spbt = spatial_emb[dist, :]
    wqt = Wq.T
    bqt = bq.reshape(D, 1)
    g2 = ln_g.reshape(1, D)
    b2 = ln_b.reshape(1, D)
    bk2 = bk.reshape(1, D)
    bv2 = bv.reshape(1, D)
    bo2 = bo.reshape(1, E)

    wspt, bspt, wvop = pl.pallas_call(
        _prep_kernel,
        out_shape=[
            jax.ShapeDtypeStruct((HL, D), jnp.float32),
            jax.ShapeDtypeStruct((HL, 1), jnp.float32),
            jax.ShapeDtypeStruct((HL, E), jnp.float32),
        ],
    )(layer_emb, cent_emb, wqt, bqt, Wk, bk2, Wv, bv2, Wo, spbt)

    wsp = wspt.T
    bsp = bspt.T

    mcol = attention_mask.reshape(B, nblk, TB, 1)

    a = pl.pallas_call(
        _main_kernel,
        grid=(B, nblk),
        in_specs=[
            pl.BlockSpec((1, TB, D), lambda bb, jj: (bb, jj, 0)),
            pl.BlockSpec((1, 1, TB, 1), lambda bb, jj: (bb, jj, 0, 0)),
            pl.BlockSpec((D, HL), lambda bb, jj: (0, 0)),
            pl.BlockSpec((1, HL), lambda bb, jj: (0, 0)),
            pl.BlockSpec((1, D), lambda bb, jj: (0, 0)),
            pl.BlockSpec((1, D), lambda bb, jj: (0, 0)),
            pl.BlockSpec((1, D), lambda bb, jj: (0, 0)),
        ],
        out_specs=pl.BlockSpec((1, 1, HL), lambda bb, jj: (bb, 0, 0)),
        out_shape=jax.ShapeDtypeStruct((B, 1, HL), jnp.float32),
    )(hidden_states, mcol, wsp, bsp, ctx_row, g2, b2)

    logits = pl.pallas_call(
        _final_kernel,
        out_shape=jax.ShapeDtypeStruct((B, E), jnp.float32),
    )(a.reshape(B, HL), attention_mask, wvop, bo2)
    return logits


def kernel(hidden_states, attention_mask, layer_emb, cent_emb, spatial_emb,
           ln_g, ln_b, Wq, bq, Wk, bk, Wv, bv, Wo, bo, position):
    return _run(hidden_states, attention_mask, layer_emb, cent_emb,
                spatial_emb, ln_g, ln_b, Wq, bq, Wk, bk, Wv, bv, Wo, bo,
                position)


# speedup vs baseline: 1.3977x; 1.0128x over previous
import functools

import jax
import jax.numpy as jnp
import numpy as np
from jax.experimental import pallas as pl
from jax.experimental.pallas import tpu as pltpu

E = 16
D = 768
H = 3
DH = D // H
NCTX = 3
MAXDIST = E
SCALE = 0.01
LANE = 128
HL = H * LANE

TB = 512


def _ln(x, g, b):
    mu = jnp.mean(x, axis=-1, keepdims=True)
    xc = x - mu
    var = jnp.mean(xc * xc, axis=-1, keepdims=True)
    return xc * jax.lax.rsqrt(var + 1e-5) * g + b


def _headwise_softmax(s):
    parts = []
    for h in range(H):
        sh = s[:, h * LANE:(h + 1) * LANE]
        mx = jnp.max(sh, axis=-1, keepdims=True)
        eh = jnp.exp(sh - mx)
        parts.append(eh / jnp.sum(eh, axis=-1, keepdims=True))
    return jnp.concatenate(parts, axis=1)


def _fused_kernel(hs, mcol, le, ce, let, cet, wq, wkt, wv, wo,
                  bq2, bkcol, bv2, spb2, g, b, ctx, bo2,
                  out_ref, wsp_s, bsp_s, wvop_s, pctx_s, a_s):
    bb = pl.program_id(0)
    jj = pl.program_id(1)
    nblk = pl.num_programs(1)
    inv = np.float32(1.0 / np.sqrt(DH))

    @pl.when(jnp.logical_and(bb == 0, jj == 0))
    def _prep():
        node = le[...] + ce[...]
        nodet = let[...] + cet[...]
        vmat = jnp.dot(node, wv[...], preferred_element_type=jnp.float32) + bv2[...]
        zlane = jnp.zeros((D, LANE - E), dtype=jnp.float32)
        zb = jnp.full((1, LANE - E), -1e30, dtype=jnp.float32)
        zvo = jnp.zeros((LANE - E, E), dtype=jnp.float32)
        wsp_parts, bsp_parts, wvop_parts = [], [], []
        for h in range(H):
            sl = slice(h * DH, (h + 1) * DH)
            kht = jnp.dot(wkt[sl, :], nodet[...],
                          preferred_element_type=jnp.float32) + bkcol[sl, :]
            ws_h = jnp.dot(wq[:, sl], kht,
                           preferred_element_type=jnp.float32) * inv
            bs_h = jnp.dot(bq2[:, sl], kht,
                           preferred_element_type=jnp.float32) * inv + spb2[...]
            wvo_h = jnp.dot(vmat[:, sl], wo[sl, :],
                            preferred_element_type=jnp.float32)
            wsp_parts += [ws_h, zlane]
            bsp_parts += [bs_h, zb]
            wvop_parts.append(jnp.concatenate([wvo_h, zvo], axis=0))
        wsp_s[...] = jnp.concatenate(wsp_parts, axis=1)
        bsp_s[...] = jnp.concatenate(bsp_parts, axis=1)
        wvop_s[...] = jnp.concatenate(wvop_parts, axis=0)
        hc = _ln(ctx[...], g[...], b[...])
        sc = jnp.dot(hc, wsp_s[...], preferred_element_type=jnp.float32) + bsp_s[...]
        pctx_s[...] = _headwise_softmax(sc)

    x = hs[0]
    hln = _ln(x, g[...], b[...])
    s = jnp.dot(hln, wsp_s[...], preferred_element_type=jnp.float32) + bsp_s[...]
    p = _headwise_softmax(s)
    m = mcol[0, 0]
    partial = jnp.sum(p * m, axis=0, keepdims=True)

    @pl.when(jj == 0)
    def _():
        a_s[...] = np.float32(NCTX) * pctx_s[...] + partial

    @pl.when(jj != 0)
    def _():
        a_s[...] = a_s[...] + partial

    @pl.when(jj == nblk - 1)
    def _final():
        a = a_s[...]
        msum = jnp.sum(a[:, :LANE], axis=-1, keepdims=True)
        acc = jnp.dot(a, wvop_s[...], preferred_element_type=jnp.float32)
        out_ref[0] = acc * (np.float32(SCALE) / jnp.maximum(msum, 1e-6)) \
            + bo2[...] * np.float32(SCALE)


@jax.jit
def _run(hidden_states, attention_mask, layer_emb, cent_emb, spatial_emb,
         ln_g, ln_b, Wq, bq, Wk, bk, Wv, bv, Wo, bo, position):
    B, S, _ = hidden_states.shape
    nblk = S // TB

    ctx_row = layer_emb[position][None, :]
    dist = jnp.clip(jnp.abs(position - jnp.arange(E)), 0, MAXDIST)
    spb2 = spatial_emb[dist, :].T
    mcol = attention_mask.reshape(B, nblk, TB, 1)

    full = lambda shp: pl.BlockSpec(shp, lambda bb, jj: tuple(0 for _ in shp))
    out = pl.pallas_call(
        _fused_kernel,
        grid=(B, nblk),
        in_specs=[
            pl.BlockSpec((1, TB, D), lambda bb, jj: (bb, jj, 0)),
            pl.BlockSpec((1, 1, TB, 1), lambda bb, jj: (bb, jj, 0, 0)),
            full((E, D)), full((E, D)), full((D, E)), full((D, E)),
            full((D, D)), full((D, D)), full((D, D)), full((D, E)),
            full((1, D)), full((D, 1)), full((1, D)), full((1, E)),
            full((1, D)), full((1, D)), full((1, D)), full((1, E)),
        ],
        out_specs=pl.BlockSpec((1, 1, E), lambda bb, jj: (bb, 0, 0)),
        out_shape=jax.ShapeDtypeStruct((B, 1, E), jnp.float32),
        scratch_shapes=[
            pltpu.VMEM((D, HL), jnp.float32),
            pltpu.VMEM((1, HL), jnp.float32),
            pltpu.VMEM((HL, E), jnp.float32),
            pltpu.VMEM((1, HL), jnp.float32),
            pltpu.VMEM((1, HL), jnp.float32),
        ],
    )(hidden_states, mcol, layer_emb, cent_emb, layer_emb.T, cent_emb.T,
      Wq, Wk.T, Wv, Wo, bq.reshape(1, D), bk.reshape(D, 1), bv.reshape(1, D),
      spb2, ln_g.reshape(1, D), ln_b.reshape(1, D), ctx_row, bo.reshape(1, E))
    return out.reshape(B, E)


def kernel(hidden_states, attention_mask, layer_emb, cent_emb, spatial_emb,
           ln_g, ln_b, Wq, bq, Wk, bk, Wv, bv, Wo, bo, position):
    return _run(hidden_states, attention_mask, layer_emb, cent_emb,
                spatial_emb, ln_g, ln_b, Wq, bq, Wk, bk, Wv, bv, Wo, bo,
                position)


# speedup vs baseline: 1.4419x; 1.0317x over previous
import functools

import jax
import jax.numpy as jnp
import numpy as np
from jax.experimental import pallas as pl
from jax.experimental.pallas import tpu as pltpu

E = 16
D = 768
H = 3
DH = D // H
NCTX = 3
MAXDIST = E
SCALE = 0.01
LANE = 128
HL = H * LANE

TB = 1024


def _ln(x, g, b):
    mu = jnp.mean(x, axis=-1, keepdims=True)
    xc = x - mu
    var = jnp.mean(xc * xc, axis=-1, keepdims=True)
    return xc * jax.lax.rsqrt(var + 1e-5) * g + b


def _headwise_softmax(s):
    parts = []
    for h in range(H):
        sh = s[:, h * LANE:(h + 1) * LANE]
        mx = jnp.max(sh, axis=-1, keepdims=True)
        eh = jnp.exp(sh - mx)
        parts.append(eh / jnp.sum(eh, axis=-1, keepdims=True))
    return jnp.concatenate(parts, axis=1)


def _fused_kernel(hs, mcol, le, ce, let, cet, wq, wkt, wv, wo,
                  bq2, bkcol, bv2, spb2, g, b, ctx, bo2,
                  out_ref, wsp_s, bsp_s, wvop_s, pctx_s, a_s):
    bb = pl.program_id(0)
    jj = pl.program_id(1)
    nblk = pl.num_programs(1)
    inv = np.float32(1.0 / np.sqrt(DH))

    @pl.when(jnp.logical_and(bb == 0, jj == 0))
    def _prep():
        node = le[...] + ce[...]
        nodet = let[...] + cet[...]
        vmat = jnp.dot(node, wv[...], preferred_element_type=jnp.float32) + bv2[...]
        zlane = jnp.zeros((D, LANE - E), dtype=jnp.float32)
        zb = jnp.full((1, LANE - E), -1e30, dtype=jnp.float32)
        zvo = jnp.zeros((LANE - E, E), dtype=jnp.float32)
        wsp_parts, bsp_parts, wvop_parts = [], [], []
        for h in range(H):
            sl = slice(h * DH, (h + 1) * DH)
            kht = jnp.dot(wkt[sl, :], nodet[...],
                          preferred_element_type=jnp.float32) + bkcol[sl, :]
            ws_h = jnp.dot(wq[:, sl], kht,
                           preferred_element_type=jnp.float32) * inv
            bs_h = jnp.dot(bq2[:, sl], kht,
                           preferred_element_type=jnp.float32) * inv + spb2[...]
            wvo_h = jnp.dot(vmat[:, sl], wo[sl, :],
                            preferred_element_type=jnp.float32)
            wsp_parts += [ws_h, zlane]
            bsp_parts += [bs_h, zb]
            wvop_parts.append(jnp.concatenate([wvo_h, zvo], axis=0))
        wsp_s[...] = jnp.concatenate(wsp_parts, axis=1)
        bsp_s[...] = jnp.concatenate(bsp_parts, axis=1)
        wvop_s[...] = jnp.concatenate(wvop_parts, axis=0)
        hc = _ln(ctx[...], g[...], b[...])
        sc = jnp.dot(hc, wsp_s[...], preferred_element_type=jnp.float32) + bsp_s[...]
        pctx_s[...] = _headwise_softmax(sc)

    x = hs[0]
    hln = _ln(x, g[...], b[...])
    s = jnp.dot(hln, wsp_s[...], preferred_element_type=jnp.float32) + bsp_s[...]
    p = _headwise_softmax(s)
    m = mcol[0, 0]
    partial = jnp.sum(p * m, axis=0, keepdims=True)

    @pl.when(jj == 0)
    def _():
        a_s[...] = np.float32(NCTX) * pctx_s[...] + partial

    @pl.when(jj != 0)
    def _():
        a_s[...] = a_s[...] + partial

    @pl.when(jj == nblk - 1)
    def _final():
        a = a_s[...]
        msum = jnp.sum(a[:, :LANE], axis=-1, keepdims=True)
        acc = jnp.dot(a, wvop_s[...], preferred_element_type=jnp.float32)
        out_ref[0] = acc * (np.float32(SCALE) / jnp.maximum(msum, 1e-6)) \
            + bo2[...] * np.float32(SCALE)


@jax.jit
def _run(hidden_states, attention_mask, layer_emb, cent_emb, spatial_emb,
         ln_g, ln_b, Wq, bq, Wk, bk, Wv, bv, Wo, bo, position):
    B, S, _ = hidden_states.shape
    nblk = S // TB

    ctx_row = layer_emb[position][None, :]
    dist = jnp.clip(jnp.abs(position - jnp.arange(E)), 0, MAXDIST)
    spb2 = spatial_emb[dist, :].T
    mcol = attention_mask.reshape(B, nblk, TB, 1)

    full = lambda shp: pl.BlockSpec(shp, lambda bb, jj: tuple(0 for _ in shp))
    out = pl.pallas_call(
        _fused_kernel,
        grid=(B, nblk),
        in_specs=[
            pl.BlockSpec((1, TB, D), lambda bb, jj: (bb, jj, 0)),
            pl.BlockSpec((1, 1, TB, 1), lambda bb, jj: (bb, jj, 0, 0)),
            full((E, D)), full((E, D)), full((D, E)), full((D, E)),
            full((D, D)), full((D, D)), full((D, D)), full((D, E)),
            full((1, D)), full((D, 1)), full((1, D)), full((1, E)),
            full((1, D)), full((1, D)), full((1, D)), full((1, E)),
        ],
        out_specs=pl.BlockSpec((1, 1, E), lambda bb, jj: (bb, 0, 0)),
        out_shape=jax.ShapeDtypeStruct((B, 1, E), jnp.float32),
        scratch_shapes=[
            pltpu.VMEM((D, HL), jnp.float32),
            pltpu.VMEM((1, HL), jnp.float32),
            pltpu.VMEM((HL, E), jnp.float32),
            pltpu.VMEM((1, HL), jnp.float32),
            pltpu.VMEM((1, HL), jnp.float32),
        ],
    )(hidden_states, mcol, layer_emb, cent_emb, layer_emb.T, cent_emb.T,
      Wq, Wk.T, Wv, Wo, bq.reshape(1, D), bk.reshape(D, 1), bv.reshape(1, D),
      spb2, ln_g.reshape(1, D), ln_b.reshape(1, D), ctx_row, bo.reshape(1, E))
    return out.reshape(B, E)


def kernel(hidden_states, attention_mask, layer_emb, cent_emb, spatial_emb,
           ln_g, ln_b, Wq, bq, Wk, bk, Wv, bv, Wo, bo, position):
    return _run(hidden_states, attention_mask, layer_emb, cent_emb,
                spatial_emb, ln_g, ln_b, Wq, bq, Wk, bk, Wv, bv, Wo, bo,
                position)
